# Initial kernel scaffold; baseline (speedup 1.0000x reference)
import jax
import jax.numpy as jnp
from jax.experimental import pallas as pl


def kernel(x, span_ranges, W_span, b_span, W_pair, b_pair):
    raise NotImplementedError("write your pallas kernel here")


# speedup vs baseline: 8.3675x; 8.3675x over previous
import functools

import jax
import jax.numpy as jnp
from jax import lax
from jax.experimental import pallas as pl
from jax.experimental.pallas import tpu as pltpu

_B, _S, _D, _C = 16, 80, 768, 3
_M = 32

_TN = (((0,), (0,)), ((), ()))
_NT = (((1,), (1,)), ((), ()))
_HI = lax.Precision.HIGHEST


def _tc_body(x_ref, hm_ref, srT_ref, w1_ref, w2_ref, w3t_ref,
             bpair_ref, logits_ref, pr_ref):
  xb = x_ref[0]
  hm_row = hm_ref[0]

  eye_s = (lax.broadcasted_iota(jnp.int32, (_S, _S), 0) ==
           lax.broadcasted_iota(jnp.int32, (_S, _S), 1)).astype(jnp.float32)
  hm_col = lax.dot_general(eye_s, hm_row, _NT, precision=_HI)

  row_i = lax.broadcasted_iota(jnp.int32, (_S, _S), 0)
  col_j = lax.broadcasted_iota(jnp.int32, (_S, _S), 1)
  beats = (hm_row > hm_col) | ((hm_row == hm_col) & (col_j < row_i))
  rank_col = jnp.sum(beats.astype(jnp.float32), axis=1, keepdims=True)

  r32_col = rank_col[:_M]
  eye_m = (lax.broadcasted_iota(jnp.int32, (_M, _M), 0) ==
           lax.broadcasted_iota(jnp.int32, (_M, _M), 1)).astype(jnp.float32)
  r32_row = lax.dot_general(r32_col, eye_m, _TN, precision=_HI)
  pos_col = jnp.sum((r32_row < r32_col).astype(jnp.float32),
                    axis=1, keepdims=True)
  oh_pos = (pos_col == lax.broadcasted_iota(
      jnp.int32, (_M, _M), 1).astype(jnp.float32)).astype(jnp.float32)
  oh_rank = (r32_col == lax.broadcasted_iota(
      jnp.int32, (_M, _S), 1).astype(jnp.float32)).astype(jnp.float32)
  P = lax.dot_general(oh_pos, oh_rank, _TN, precision=_HI)

  x_rk = jnp.dot(P, xb, precision=_HI)

  A = jnp.dot(x_rk, w1_ref[...], precision=_HI)
  Bt = lax.dot_general(w2_ref[...], x_rk, (((0,), (1,)), ((), ())),
                       precision=_HI)
  bp = bpair_ref[...]
  sig = []
  for c in range(_C):
    wc = w3t_ref[c:c + 1, :]
    Mc = lax.dot_general(x_rk * wc, x_rk, _NT, precision=_HI)
    pre = Mc + A[:, c:c + 1] + Bt[c:c + 1, :] + bp[0, c]
    sig.append(jax.nn.sigmoid(pre))
  mx = jnp.maximum(jnp.maximum(sig[0], sig[1]), sig[2])
  es = [jnp.exp(s - mx) for s in sig]
  den = es[0] + es[1] + es[2]
  for c in range(_C):
    logits_ref[0, c] = es[c] / den

  Pi = P.astype(jnp.int32)
  sr0 = jnp.sum(Pi * srT_ref[0:1, :], axis=1, keepdims=True)
  sr1 = jnp.sum(Pi * srT_ref[1:2, :], axis=1, keepdims=True)
  sr0_row = lax.dot_general(sr0.astype(jnp.float32), eye_m, _TN,
                            precision=_HI).astype(jnp.int32)
  sr1_row = lax.dot_general(sr1.astype(jnp.float32), eye_m, _TN,
                            precision=_HI).astype(jnp.int32)
  zero_m = jnp.zeros((_M, _M), jnp.int32)
  pr_ref[0, 0] = zero_m + sr0
  pr_ref[0, 1] = zero_m + sr1
  pr_ref[0, 2] = zero_m + sr0_row
  pr_ref[0, 3] = zero_m + sr1_row


@functools.partial(jax.jit, static_argnames=("interpret",))
def _run(x, span_ranges, W_span, b_span, W_pair, b_pair, interpret=False):
  srT = span_ranges.T
  W1 = W_pair[:_D, :]
  W2 = W_pair[_D:2 * _D, :]
  W3T = W_pair[2 * _D:, :].T
  bpair = b_pair.reshape(1, _C)
  hm = jax.nn.sigmoid(x @ W_span + b_span).mean(axis=-1)
  hm3 = hm.reshape(_B, 1, _S)

  grid = (_B,)
  logits_full, pr_full = pl.pallas_call(
      _tc_body,
      grid=grid,
      in_specs=[
          pl.BlockSpec((1, _S, _D), lambda b: (b, 0, 0)),
          pl.BlockSpec((1, 1, _S), lambda b: (b, 0, 0)),
          pl.BlockSpec((2, _S), lambda b: (0, 0)),
          pl.BlockSpec((_D, _C), lambda b: (0, 0)),
          pl.BlockSpec((_D, _C), lambda b: (0, 0)),
          pl.BlockSpec((_C, _D), lambda b: (0, 0)),
          pl.BlockSpec((1, _C), lambda b: (0, 0)),
      ],
      out_specs=[
          pl.BlockSpec((1, _C, _M, _M), lambda b: (b, 0, 0, 0)),
          pl.BlockSpec((1, 4, _M, _M), lambda b: (b, 0, 0, 0)),
      ],
      out_shape=[
          jax.ShapeDtypeStruct((_B, _C, _M, _M), jnp.float32),
          jax.ShapeDtypeStruct((_B, 4, _M, _M), jnp.int32),
      ],
      interpret=interpret,
  )(x, hm3, srT, W1, W2, W3T, bpair)

  n_off = _M * (_M - 1)
  logits = logits_full.reshape(_B, _C, _M * _M).transpose(0, 2, 1)
  logits = logits[:, 1:, :].reshape(_B, _M - 1, _M + 1, _C)[:, :, :_M, :]
  logits = logits.reshape(_B, n_off, _C)
  pr = pr_full.reshape(_B, 4, _M * _M).transpose(0, 2, 1)
  pr = pr[:, 1:, :].reshape(_B, _M - 1, _M + 1, 4)[:, :, :_M, :]
  pr = pr.reshape(_B, n_off, 2, 2)
  return logits, pr


def kernel(x, span_ranges, W_span, b_span, W_pair, b_pair):
  return _run(x, span_ranges, W_span, b_span, W_pair, b_pair)
